# Initial kernel scaffold; baseline (speedup 1.0000x reference)
import jax
import jax.numpy as jnp
from jax.experimental import pallas as pl


def kernel(xyz, params):
    raise NotImplementedError("write your pallas kernel here")


# speedup vs baseline: 4.2172x; 4.2172x over previous
import functools
from typing import Sequence

import jax
import jax.numpy as jnp
from jax.experimental import pallas as pl

F32 = jnp.float32
I32 = jnp.int32
BIG = 1e30


def _first_min_index(vals, iota, axis):
    m = jnp.min(vals, axis=axis, keepdims=True)
    big_i = jnp.array(2**30, I32)
    return jnp.min(jnp.where(vals == m, iota, big_i), axis=axis, keepdims=True)


def _first_max_index(vals, iota, axis):
    m = jnp.max(vals, axis=axis, keepdims=True)
    big_i = jnp.array(2**30, I32)
    return jnp.min(jnp.where(vals == m, iota, big_i), axis=axis, keepdims=True)


def _fps_body(xyzT_ref, out_ref, *, n_valid, n_center, ncp):
    xyzT = xyzT_ref[0]
    np_ = xyzT.shape[1]
    lane = jax.lax.broadcasted_iota(I32, (1, np_), 1)
    valid = lane < n_valid
    out_lane = jax.lax.broadcasted_iota(I32, (1, ncp), 1)

    dists0 = jnp.where(valid, jnp.full((1, np_), 1e10, F32), -1.0)
    far0 = jnp.zeros((1, 1), I32)
    acc0 = jnp.zeros((3, ncp), F32)

    def body(i, carry):
        dists, far, acc = carry
        oh = (lane == far).astype(F32)
        c = jnp.sum(xyzT * oh, axis=1, keepdims=True)
        acc = jnp.where(out_lane == i, c, acc)
        d = jnp.sum((xyzT - c) ** 2, axis=0, keepdims=True)
        dists = jnp.where(valid, jnp.minimum(dists, d), -1.0)
        far = _first_max_index(dists, lane, axis=1)
        return dists, far, acc

    _, _, acc = jax.lax.fori_loop(0, n_center, body, (dists0, far0, acc0))
    out_ref[0] = acc


def _fps(xyzT, n_valid, n_center, ncp):
    b = xyzT.shape[0]
    np_ = xyzT.shape[2]
    return pl.pallas_call(
        functools.partial(_fps_body, n_valid=n_valid, n_center=n_center,
                          ncp=ncp),
        grid=(b,),
        in_specs=[pl.BlockSpec((1, 3, np_), lambda i: (i, 0, 0))],
        out_specs=pl.BlockSpec((1, 3, ncp), lambda i: (i, 0, 0)),
        out_shape=jax.ShapeDtypeStruct((b, 3, ncp), F32),
    )(xyzT)


def _knn_body(q_ref, refT_ref, idx_ref, *, n_valid, k):
    q = q_ref[0]
    refT = refT_ref[0]
    t, np_ = q.shape[0], refT.shape[1]
    qr = jax.lax.dot_general(q, refT, (((1,), (0,)), ((), ())),
                             preferred_element_type=F32)
    q2 = jnp.sum(q * q, axis=1, keepdims=True)
    r2 = jnp.sum(refT * refT, axis=0, keepdims=True)
    d = q2 - 2.0 * qr + r2
    lane = jax.lax.broadcasted_iota(I32, (t, np_), 1)
    d = jnp.where(lane < n_valid, d, BIG)
    k_lane = jax.lax.broadcasted_iota(I32, (t, k), 1)
    acc0 = jnp.zeros((t, k), I32)

    def body(j, carry):
        d, acc = carry
        sel = _first_min_index(d, lane, axis=1)
        acc = jnp.where(k_lane == j, sel, acc)
        d = jnp.where(lane == sel, BIG, d)
        return d, acc

    _, acc = jax.lax.fori_loop(0, k, body, (d, acc0))
    idx_ref[0] = acc


def _knn(query, refT, n_valid, k, t):
    b, s, _ = query.shape
    np_ = refT.shape[2]
    return pl.pallas_call(
        functools.partial(_knn_body, n_valid=n_valid, k=k),
        grid=(b, s // t),
        in_specs=[
            pl.BlockSpec((1, t, 3), lambda i, j: (i, j, 0)),
            pl.BlockSpec((1, 3, np_), lambda i, j: (i, 0, 0)),
        ],
        out_specs=pl.BlockSpec((1, t, k), lambda i, j: (i, j, 0)),
        out_shape=jax.ShapeDtypeStruct((b, s, k), I32),
    )(query, refT)


def _sa_body(idx_ref, table_ref, ctr_ref, *rest, k, n_layers):
    wrefs = rest[:-1]
    out_ref = rest[-1]
    idx = idx_ref[0]
    table = table_ref[0]
    ctr = ctr_ref[0]
    tk = idx.shape[0]
    np_ = table.shape[0]
    t = ctr.shape[0]
    cp = table.shape[1] - 3

    lane = jax.lax.broadcasted_iota(I32, (tk, np_), 1)
    oh = (lane == idx).astype(F32)
    g = jax.lax.dot_general(oh, table, (((1,), (0,)), ((), ())),
                            preferred_element_type=F32)

    seg_row = jax.lax.broadcasted_iota(I32, (tk, t), 0) // k
    prow = (seg_row == jax.lax.broadcasted_iota(I32, (tk, t), 1)).astype(F32)
    seg_col = jax.lax.broadcasted_iota(I32, (t, tk), 1) // k
    psum = (jax.lax.broadcasted_iota(I32, (t, tk), 0) == seg_col).astype(F32)

    cxyz = jax.lax.dot_general(prow, ctr, (((1,), (0,)), ((), ())),
                               preferred_element_type=F32)
    gxyz = g[:, cp:] - cxyz
    feat = jnp.concatenate([g[:, :cp], gxyz], axis=1)

    for li in range(n_layers):
        wt, bgbe = wrefs[2 * li][...], wrefs[2 * li + 1][...]
        f = jax.lax.dot_general(feat, wt, (((1,), (0,)), ((), ())),
                                preferred_element_type=F32)
        f = f + bgbe[0:1]
        f = bgbe[1:2] * f + bgbe[2:3]
        feat = jnp.maximum(f, 0.0)

    att = wrefs[2 * n_layers][...]
    att_b = wrefs[2 * n_layers + 1][...]
    scores = jax.lax.dot_general(feat, att, (((1,), (0,)), ((), ())),
                                 preferred_element_type=F32) + att_b
    m = jnp.max(scores, axis=0, keepdims=True)
    e = jnp.exp(scores - m)
    seg_sum = jax.lax.dot_general(psum, e, (((1,), (0,)), ((), ())),
                                  preferred_element_type=F32)
    denom = jax.lax.dot_general(prow, seg_sum, (((1,), (0,)), ((), ())),
                                preferred_element_type=F32)
    attn = e / denom
    pooled = jax.lax.dot_general(psum, attn * feat, (((1,), (0,)), ((), ())),
                                 preferred_element_type=F32)
    out_ref[0] = pooled


def _sa(idx_flat, table, centers, layers, att_w, att_b, k, t):
    b, s, _ = centers.shape
    np_, ct = table.shape[1], table.shape[2]
    tk = t * k
    cout = layers[-1]["W"].shape[0]
    wargs, wspecs = [], []
    for L in layers:
        wt = jnp.transpose(L["W"])
        bgbe = jnp.stack([L["b"], L["g"], L["be"]])
        wargs += [wt, bgbe]
        wspecs += [pl.BlockSpec(wt.shape, lambda i, j: (0, 0)),
                   pl.BlockSpec(bgbe.shape, lambda i, j: (0, 0))]
    attw_t = jnp.transpose(att_w)
    attb = att_b.reshape(1, 1)
    wargs += [attw_t, attb]
    wspecs += [pl.BlockSpec(attw_t.shape, lambda i, j: (0, 0)),
               pl.BlockSpec((1, 1), lambda i, j: (0, 0))]
    return pl.pallas_call(
        functools.partial(_sa_body, k=k, n_layers=len(layers)),
        grid=(b, s // t),
        in_specs=[
            pl.BlockSpec((1, tk, 1), lambda i, j: (i, j, 0)),
            pl.BlockSpec((1, np_, ct), lambda i, j: (i, 0, 0)),
            pl.BlockSpec((1, t, 3), lambda i, j: (i, j, 0)),
        ] + wspecs,
        out_specs=pl.BlockSpec((1, t, cout), lambda i, j: (i, j, 0)),
        out_shape=jax.ShapeDtypeStruct((b, s, cout), F32),
    )(idx_flat, table, centers, *wargs)


def _fp_body(q_ref, refT_ref, table_ref, pts1_ref, *rest, n_valid, relu_flags):
    n_layers = len(relu_flags)
    wrefs = rest[:-1]
    out_ref = rest[-1]
    q = q_ref[0]
    refT = refT_ref[0]
    table = table_ref[0]
    pts1 = pts1_ref[0]
    t, np_ = q.shape[0], refT.shape[1]
    c2 = table.shape[1] - 3

    qr = jax.lax.dot_general(q, refT, (((1,), (0,)), ((), ())),
                             preferred_element_type=F32)
    q2 = jnp.sum(q * q, axis=1, keepdims=True)
    r2 = jnp.sum(refT * refT, axis=0, keepdims=True)
    d = q2 - 2.0 * qr + r2
    lane = jax.lax.broadcasted_iota(I32, (t, np_), 1)
    d = jnp.where(lane < n_valid, d, BIG)

    ws, gs = [], []
    for _ in range(3):
        sel = _first_min_index(d, lane, axis=1)
        d = jnp.where(lane == sel, BIG, d)
        oh = (lane == sel).astype(F32)
        gj = jax.lax.dot_general(oh, table, (((1,), (0,)), ((), ())),
                                 preferred_element_type=F32)
        dj = jnp.sum((q - gj[:, c2:]) ** 2, axis=1, keepdims=True)
        ws.append(1.0 / (dj + 1e-8))
        gs.append(gj[:, :c2])
    wsum = ws[0] + ws[1] + ws[2]
    interp = (ws[0] * gs[0] + ws[1] * gs[1] + ws[2] * gs[2]) / wsum
    feat = jnp.concatenate([pts1, interp], axis=1)

    for li in range(n_layers):
        wt, bgbe = wrefs[2 * li][...], wrefs[2 * li + 1][...]
        f = jax.lax.dot_general(feat, wt, (((1,), (0,)), ((), ())),
                                preferred_element_type=F32)
        f = f + bgbe[0:1]
        f = bgbe[1:2] * f + bgbe[2:3]
        feat = jnp.maximum(f, 0.0) if relu_flags[li] else f
    out_ref[0] = feat


def _fp(query, refT, table, pts1, layer_list, relu_flags, n_valid, t):
    b, s, _ = query.shape
    np_, ct = table.shape[1], table.shape[2]
    c1 = pts1.shape[2]
    cout = layer_list[-1][0].shape[1]
    wargs, wspecs = [], []
    for wt, bgbe in layer_list:
        wargs += [wt, bgbe]
        wspecs += [pl.BlockSpec(wt.shape, lambda i, j: (0, 0)),
                   pl.BlockSpec(bgbe.shape, lambda i, j: (0, 0))]
    return pl.pallas_call(
        functools.partial(_fp_body, n_valid=n_valid,
                          relu_flags=tuple(relu_flags)),
        grid=(b, s // t),
        in_specs=[
            pl.BlockSpec((1, t, 3), lambda i, j: (i, j, 0)),
            pl.BlockSpec((1, 3, np_), lambda i, j: (i, 0, 0)),
            pl.BlockSpec((1, np_, ct), lambda i, j: (i, 0, 0)),
            pl.BlockSpec((1, t, c1), lambda i, j: (i, j, 0)),
        ] + wspecs,
        out_specs=pl.BlockSpec((1, t, cout), lambda i, j: (i, j, 0)),
        out_shape=jax.ShapeDtypeStruct((b, s, cout), F32),
    )(query, refT, table, pts1, *wargs)


def _mk_layers(mlp):
    return [(jnp.transpose(L["W"]), jnp.stack([L["b"], L["g"], L["be"]]))
            for L in mlp]


def kernel(xyz, params):
    b, n, _ = xyz.shape
    nc1, nn1 = 921, 50
    nc2, nn2 = 829, 100
    s1p, s2p = 928, 832

    p = params
    xyzT = jnp.transpose(xyz, (0, 2, 1))

    l1T = _fps(xyzT, n_valid=n, n_center=nc1, ncp=s1p)
    l1_xyz = jnp.transpose(l1T, (0, 2, 1))
    idx1 = _knn(l1_xyz, xyzT, n_valid=n, k=nn1, t=464)
    idx1f = idx1.reshape(b, s1p * nn1, 1)
    table1 = jnp.concatenate([xyz, xyz], axis=2)
    l1_pts = _sa(idx1f, table1, l1_xyz, p["sa1"]["mlp"],
                 p["sa1"]["att_W"], p["sa1"]["att_b"], k=nn1, t=32)

    l2T = _fps(l1T, n_valid=nc1, n_center=nc2, ncp=s2p)
    l2_xyz = jnp.transpose(l2T, (0, 2, 1))
    l1xT = jnp.transpose(l1_xyz, (0, 2, 1))
    idx2 = _knn(l2_xyz, l1xT, n_valid=nc1, k=nn2, t=416)
    idx2f = idx2.reshape(b, s2p * nn2, 1)
    table2 = jnp.concatenate([l1_pts, l1_xyz], axis=2)
    l2_pts = _sa(idx2f, table2, l2_xyz, p["sa2"]["mlp"],
                 p["sa2"]["att_W"], p["sa2"]["att_b"], k=nn2, t=32)

    l2xT = jnp.transpose(l2_xyz, (0, 2, 1))
    ftable2 = jnp.concatenate([l2_pts, l2_xyz], axis=2)
    l1_new = _fp(l1_xyz, l2xT, ftable2, l1_pts,
                 _mk_layers(p["fp2"]["mlp"]), [True, True, True],
                 n_valid=nc2, t=464)

    ftable1 = jnp.concatenate([l1_new, l1_xyz], axis=2)
    pts0 = jnp.concatenate([xyz, xyz], axis=2)
    ones = jnp.ones_like(p["conv2_b"])
    zeros = jnp.zeros_like(p["conv2_b"])
    head = _mk_layers(p["fp1"]["mlp"]) + [
        (jnp.transpose(p["conv1_W"]),
         jnp.stack([p["conv1_b"], p["bn1_g"], p["bn1_b"]])),
        (jnp.transpose(p["conv2_W"]),
         jnp.stack([p["conv2_b"], ones, zeros])),
    ]
    out = _fp(xyz, l1xT, ftable1, pts0, head,
              [True, True, True, True, False], n_valid=nc1, t=512)
    return out


# speedup vs baseline: 6.5215x; 1.5464x over previous
import functools
from typing import Sequence

import jax
import jax.numpy as jnp
from jax.experimental import pallas as pl

F32 = jnp.float32
I32 = jnp.int32
BIG = 1e30


def _first_min_index(vals, iota, axis):
    m = jnp.min(vals, axis=axis, keepdims=True)
    big_i = jnp.array(2**30, I32)
    return jnp.min(jnp.where(vals == m, iota, big_i), axis=axis, keepdims=True)


def _first_max_index(vals, iota, axis):
    m = jnp.max(vals, axis=axis, keepdims=True)
    big_i = jnp.array(2**30, I32)
    return jnp.min(jnp.where(vals == m, iota, big_i), axis=axis, keepdims=True)


def _fps_body(xyzT_ref, out_ref, *, n_valid, n_center, ncp):
    x = xyzT_ref[...]
    b, _, np_ = x.shape
    lane = jax.lax.broadcasted_iota(I32, (b, 1, np_), 2)
    valid = lane < n_valid
    out_lane = jax.lax.broadcasted_iota(I32, (b, 3, ncp), 2)

    dists0 = jnp.where(valid, jnp.full((b, 1, np_), 1e10, F32), -1.0)
    far0 = jnp.zeros((b, 1, 1), I32)
    acc0 = jnp.zeros((b, 3, ncp), F32)

    def body(i, carry):
        dists, far, acc = carry
        oh = (lane == far).astype(F32)
        c = jnp.sum(x * oh, axis=2, keepdims=True)
        acc = jnp.where(out_lane == i, c, acc)
        d = jnp.sum((x - c) ** 2, axis=1, keepdims=True)
        dists = jnp.where(valid, jnp.minimum(dists, d), -1.0)
        far = _first_max_index(dists, lane, axis=2)
        return dists, far, acc

    _, _, acc = jax.lax.fori_loop(0, n_center, body, (dists0, far0, acc0))
    out_ref[...] = acc


def _fps(xyzT, n_valid, n_center, ncp):
    b = xyzT.shape[0]
    return pl.pallas_call(
        functools.partial(_fps_body, n_valid=n_valid, n_center=n_center,
                          ncp=ncp),
        out_shape=jax.ShapeDtypeStruct((b, 3, ncp), F32),
    )(xyzT)


def _knn_body(q_ref, refT_ref, idx_ref, *, n_valid, k):
    q = q_ref[0]
    refT = refT_ref[0]
    t, np_ = q.shape[0], refT.shape[1]
    qr = jax.lax.dot_general(q, refT, (((1,), (0,)), ((), ())),
                             preferred_element_type=F32)
    q2 = jnp.sum(q * q, axis=1, keepdims=True)
    r2 = jnp.sum(refT * refT, axis=0, keepdims=True)
    d = q2 - 2.0 * qr + r2
    lane = jax.lax.broadcasted_iota(I32, (t, np_), 1)
    d = jnp.where(lane < n_valid, d, BIG)
    k_lane = jax.lax.broadcasted_iota(I32, (t, k), 1)
    acc0 = jnp.zeros((t, k), I32)

    def body(j, carry):
        d, acc = carry
        sel = _first_min_index(d, lane, axis=1)
        acc = jnp.where(k_lane == j, sel, acc)
        d = jnp.where(lane == sel, BIG, d)
        return d, acc

    _, acc = jax.lax.fori_loop(0, k, body, (d, acc0))
    idx_ref[0] = acc


def _knn(query, refT, n_valid, k, t):
    b, s, _ = query.shape
    np_ = refT.shape[2]
    return pl.pallas_call(
        functools.partial(_knn_body, n_valid=n_valid, k=k),
        grid=(b, s // t),
        in_specs=[
            pl.BlockSpec((1, t, 3), lambda i, j: (i, j, 0)),
            pl.BlockSpec((1, 3, np_), lambda i, j: (i, 0, 0)),
        ],
        out_specs=pl.BlockSpec((1, t, k), lambda i, j: (i, j, 0)),
        out_shape=jax.ShapeDtypeStruct((b, s, k), I32),
    )(query, refT)


def _sa_body(idx_ref, table_ref, ctr_ref, *rest, k, n_layers):
    wrefs = rest[:-1]
    out_ref = rest[-1]
    idx = idx_ref[0]
    table = table_ref[0]
    ctr = ctr_ref[0]
    tk = idx.shape[0]
    np_ = table.shape[0]
    t = ctr.shape[0]
    cp = table.shape[1] - 3

    lane = jax.lax.broadcasted_iota(I32, (tk, np_), 1)
    oh = (lane == idx).astype(F32)
    g = jax.lax.dot_general(oh, table, (((1,), (0,)), ((), ())),
                            preferred_element_type=F32)

    seg_row = jax.lax.broadcasted_iota(I32, (tk, t), 0) // k
    prow = (seg_row == jax.lax.broadcasted_iota(I32, (tk, t), 1)).astype(F32)
    seg_col = jax.lax.broadcasted_iota(I32, (t, tk), 1) // k
    psum = (jax.lax.broadcasted_iota(I32, (t, tk), 0) == seg_col).astype(F32)

    cxyz = jax.lax.dot_general(prow, ctr, (((1,), (0,)), ((), ())),
                               preferred_element_type=F32)
    gxyz = g[:, cp:] - cxyz
    feat = jnp.concatenate([g[:, :cp], gxyz], axis=1)

    for li in range(n_layers):
        wt, bgbe = wrefs[2 * li][...], wrefs[2 * li + 1][...]
        f = jax.lax.dot_general(feat, wt, (((1,), (0,)), ((), ())),
                                preferred_element_type=F32)
        f = f + bgbe[0:1]
        f = bgbe[1:2] * f + bgbe[2:3]
        feat = jnp.maximum(f, 0.0)

    att = wrefs[2 * n_layers][...]
    att_b = wrefs[2 * n_layers + 1][...]
    scores = jax.lax.dot_general(feat, att, (((1,), (0,)), ((), ())),
                                 preferred_element_type=F32) + att_b
    m = jnp.max(scores, axis=0, keepdims=True)
    e = jnp.exp(scores - m)
    seg_sum = jax.lax.dot_general(psum, e, (((1,), (0,)), ((), ())),
                                  preferred_element_type=F32)
    denom = jax.lax.dot_general(prow, seg_sum, (((1,), (0,)), ((), ())),
                                preferred_element_type=F32)
    attn = e / denom
    pooled = jax.lax.dot_general(psum, attn * feat, (((1,), (0,)), ((), ())),
                                 preferred_element_type=F32)
    out_ref[0] = pooled


def _sa(idx_flat, table, centers, layers, att_w, att_b, k, t):
    b, s, _ = centers.shape
    np_, ct = table.shape[1], table.shape[2]
    tk = t * k
    cout = layers[-1]["W"].shape[0]
    wargs, wspecs = [], []
    for L in layers:
        wt = jnp.transpose(L["W"])
        bgbe = jnp.stack([L["b"], L["g"], L["be"]])
        wargs += [wt, bgbe]
        wspecs += [pl.BlockSpec(wt.shape, lambda i, j: (0, 0)),
                   pl.BlockSpec(bgbe.shape, lambda i, j: (0, 0))]
    attw_t = jnp.transpose(att_w)
    attb = att_b.reshape(1, 1)
    wargs += [attw_t, attb]
    wspecs += [pl.BlockSpec(attw_t.shape, lambda i, j: (0, 0)),
               pl.BlockSpec((1, 1), lambda i, j: (0, 0))]
    return pl.pallas_call(
        functools.partial(_sa_body, k=k, n_layers=len(layers)),
        grid=(b, s // t),
        in_specs=[
            pl.BlockSpec((1, tk, 1), lambda i, j: (i, j, 0)),
            pl.BlockSpec((1, np_, ct), lambda i, j: (i, 0, 0)),
            pl.BlockSpec((1, t, 3), lambda i, j: (i, j, 0)),
        ] + wspecs,
        out_specs=pl.BlockSpec((1, t, cout), lambda i, j: (i, j, 0)),
        out_shape=jax.ShapeDtypeStruct((b, s, cout), F32),
    )(idx_flat, table, centers, *wargs)


def _fp_body(q_ref, refT_ref, table_ref, pts1_ref, *rest, n_valid, relu_flags):
    n_layers = len(relu_flags)
    wrefs = rest[:-1]
    out_ref = rest[-1]
    q = q_ref[0]
    refT = refT_ref[0]
    table = table_ref[0]
    pts1 = pts1_ref[0]
    t, np_ = q.shape[0], refT.shape[1]
    c2 = table.shape[1] - 3

    qr = jax.lax.dot_general(q, refT, (((1,), (0,)), ((), ())),
                             preferred_element_type=F32)
    q2 = jnp.sum(q * q, axis=1, keepdims=True)
    r2 = jnp.sum(refT * refT, axis=0, keepdims=True)
    d = q2 - 2.0 * qr + r2
    lane = jax.lax.broadcasted_iota(I32, (t, np_), 1)
    d = jnp.where(lane < n_valid, d, BIG)

    ws, gs = [], []
    for _ in range(3):
        sel = _first_min_index(d, lane, axis=1)
        d = jnp.where(lane == sel, BIG, d)
        oh = (lane == sel).astype(F32)
        gj = jax.lax.dot_general(oh, table, (((1,), (0,)), ((), ())),
                                 preferred_element_type=F32)
        dj = jnp.sum((q - gj[:, c2:]) ** 2, axis=1, keepdims=True)
        ws.append(1.0 / (dj + 1e-8))
        gs.append(gj[:, :c2])
    wsum = ws[0] + ws[1] + ws[2]
    interp = (ws[0] * gs[0] + ws[1] * gs[1] + ws[2] * gs[2]) / wsum
    feat = jnp.concatenate([pts1, interp], axis=1)

    for li in range(n_layers):
        wt, bgbe = wrefs[2 * li][...], wrefs[2 * li + 1][...]
        f = jax.lax.dot_general(feat, wt, (((1,), (0,)), ((), ())),
                                preferred_element_type=F32)
        f = f + bgbe[0:1]
        f = bgbe[1:2] * f + bgbe[2:3]
        feat = jnp.maximum(f, 0.0) if relu_flags[li] else f
    out_ref[0] = feat


def _fp(query, refT, table, pts1, layer_list, relu_flags, n_valid, t):
    b, s, _ = query.shape
    np_, ct = table.shape[1], table.shape[2]
    c1 = pts1.shape[2]
    cout = layer_list[-1][0].shape[1]
    wargs, wspecs = [], []
    for wt, bgbe in layer_list:
        wargs += [wt, bgbe]
        wspecs += [pl.BlockSpec(wt.shape, lambda i, j: (0, 0)),
                   pl.BlockSpec(bgbe.shape, lambda i, j: (0, 0))]
    return pl.pallas_call(
        functools.partial(_fp_body, n_valid=n_valid,
                          relu_flags=tuple(relu_flags)),
        grid=(b, s // t),
        in_specs=[
            pl.BlockSpec((1, t, 3), lambda i, j: (i, j, 0)),
            pl.BlockSpec((1, 3, np_), lambda i, j: (i, 0, 0)),
            pl.BlockSpec((1, np_, ct), lambda i, j: (i, 0, 0)),
            pl.BlockSpec((1, t, c1), lambda i, j: (i, j, 0)),
        ] + wspecs,
        out_specs=pl.BlockSpec((1, t, cout), lambda i, j: (i, j, 0)),
        out_shape=jax.ShapeDtypeStruct((b, s, cout), F32),
    )(query, refT, table, pts1, *wargs)


def _mk_layers(mlp):
    return [(jnp.transpose(L["W"]), jnp.stack([L["b"], L["g"], L["be"]]))
            for L in mlp]


def kernel(xyz, params):
    b, n, _ = xyz.shape
    nc1, nn1 = 921, 50
    nc2, nn2 = 829, 100
    s1p, s2p = 928, 832

    p = params
    xyzT = jnp.transpose(xyz, (0, 2, 1))

    l1T = _fps(xyzT, n_valid=n, n_center=nc1, ncp=s1p)
    l1_xyz = jnp.transpose(l1T, (0, 2, 1))
    idx1 = _knn(l1_xyz, xyzT, n_valid=n, k=nn1, t=464)
    idx1f = idx1.reshape(b, s1p * nn1, 1)
    table1 = jnp.concatenate([xyz, xyz], axis=2)
    l1_pts = _sa(idx1f, table1, l1_xyz, p["sa1"]["mlp"],
                 p["sa1"]["att_W"], p["sa1"]["att_b"], k=nn1, t=32)

    l2T = _fps(l1T, n_valid=nc1, n_center=nc2, ncp=s2p)
    l2_xyz = jnp.transpose(l2T, (0, 2, 1))
    l1xT = jnp.transpose(l1_xyz, (0, 2, 1))
    idx2 = _knn(l2_xyz, l1xT, n_valid=nc1, k=nn2, t=416)
    idx2f = idx2.reshape(b, s2p * nn2, 1)
    table2 = jnp.concatenate([l1_pts, l1_xyz], axis=2)
    l2_pts = _sa(idx2f, table2, l2_xyz, p["sa2"]["mlp"],
                 p["sa2"]["att_W"], p["sa2"]["att_b"], k=nn2, t=32)

    l2xT = jnp.transpose(l2_xyz, (0, 2, 1))
    ftable2 = jnp.concatenate([l2_pts, l2_xyz], axis=2)
    l1_new = _fp(l1_xyz, l2xT, ftable2, l1_pts,
                 _mk_layers(p["fp2"]["mlp"]), [True, True, True],
                 n_valid=nc2, t=464)

    ftable1 = jnp.concatenate([l1_new, l1_xyz], axis=2)
    pts0 = jnp.concatenate([xyz, xyz], axis=2)
    ones = jnp.ones_like(p["conv2_b"])
    zeros = jnp.zeros_like(p["conv2_b"])
    head = _mk_layers(p["fp1"]["mlp"]) + [
        (jnp.transpose(p["conv1_W"]),
         jnp.stack([p["conv1_b"], p["bn1_g"], p["bn1_b"]])),
        (jnp.transpose(p["conv2_W"]),
         jnp.stack([p["conv2_b"], ones, zeros])),
    ]
    out = _fp(xyz, l1xT, ftable1, pts0, head,
              [True, True, True, True, False], n_valid=nc1, t=512)
    return out
